# Initial kernel scaffold; baseline (speedup 1.0000x reference)
import jax
import jax.numpy as jnp
from jax.experimental import pallas as pl


def kernel(features, W_conv, b_conv, W_obj, b_obj, W_delta, b_delta):
    raise NotImplementedError("write your pallas kernel here")


# speedup vs baseline: 48.6524x; 48.6524x over previous
import jax
import jax.numpy as jnp
import numpy as np
from jax import lax
from jax.experimental import pallas as pl
from jax.experimental.pallas import tpu as pltpu

C = 256
A = 3
H = 80
W = 80
STRIDE = 8
IMG = 640.0
PRE_NMS_TOPK = 2000
POST_NMS_TOPK = 1000
NMS_THRESH = 0.7
SCALE_CLAMP = float(np.log(1000.0 / 16.0))

NP = H * W
BLK = 640
PADR = 128
NANCH = NP * A
NSORT = 32768
NTOP = 2048
NEG = -3.0e38
TAPS = [(kh, kw) for kh in range(3) for kw in range(3)]
DEF = lax.Precision.DEFAULT


def _rotations(x, j):
    R, L = x.shape
    if j >= L:
        r = j // L
        down = jnp.concatenate([x[r:], x[:r]], axis=0)
        up = jnp.concatenate([x[R - r:], x[:R - r]], axis=0)
    else:
        down = jnp.concatenate([x[:, j:], x[:, :j]], axis=1)
        up = jnp.concatenate([x[:, L - j:], x[:, :L - j]], axis=1)
    return down, up


def _bitonic_sort(key, payloads, descending):
    R, L = key.shape
    n = R * L
    levels = int(np.log2(n))
    assert 1 << levels == n
    idx = (lax.broadcasted_iota(jnp.int32, (R, L), 0) * L
           + lax.broadcasted_iota(jnp.int32, (R, L), 1))
    for lev in range(1, levels + 1):
        K = 1 << lev
        for je in range(lev - 1, -1, -1):
            j = 1 << je
            bit = (idx & j) != 0
            kd, ku = _rotations(key, j)
            pkey = jnp.where(bit, ku, kd)
            if descending:
                blockdesc = (idx & K) == 0
            else:
                blockdesc = (idx & K) != 0
            take_max = bit != blockdesc
            take_partner = ((take_max & (pkey > key))
                            | (~take_max & (pkey < key)))
            key = jnp.where(take_partner, pkey, key)
            new_payloads = []
            for p in payloads:
                pd, pu = _rotations(p, j)
                pp = jnp.where(bit, pu, pd)
                new_payloads.append(jnp.where(take_partner, pp, p))
            payloads = new_payloads
    return key, payloads


def _k1_body(xpad_ref, w_ref, b_ref, wo_ref, bo_ref, wdx_ref, wdy_ref,
             wdw_ref, wdh_ref, bdx_ref, bdy_ref, bdw_ref, bdh_ref,
             sc_ref, x1_ref, y1_ref, x2_ref, y2_ref):
    pid = pl.program_id(0)
    p0 = pid * BLK
    gp = p0 + lax.broadcasted_iota(jnp.int32, (BLK, 1), 0)
    hh1 = gp // W
    ww1 = gp % W
    xwin = xpad_ref[pl.ds(p0 + PADR - 88, BLK + 176), :]
    shifts = []
    for i, (kh, kw) in enumerate(TAPS):
        dh = kh - 1
        dw = kw - 1
        t = dh * W + dw
        xs = xwin[88 + t: 88 + t + BLK]
        valid = ((hh1 + dh >= 0) & (hh1 + dh < H)
                 & (ww1 + dw >= 0) & (ww1 + dw < W))
        shifts.append(xs * valid.astype(jnp.float32))
    x2 = jnp.concatenate(shifts, axis=1)
    acc = jnp.dot(x2, w_ref[...].reshape(9 * C, C),
                  preferred_element_type=jnp.float32, precision=DEF)
    t_act = jnp.maximum(acc + b_ref[...], 0.0)

    def head(wr, br):
        return jnp.dot(t_act, wr[...], preferred_element_type=jnp.float32,
                       precision=DEF) + br[...]

    logits = head(wo_ref, bo_ref)
    dx = head(wdx_ref, bdx_ref)
    dy = head(wdy_ref, bdy_ref)
    dwv = head(wdw_ref, bdw_ref)
    dhv = head(wdh_ref, bdh_ref)

    pix = p0 + lax.broadcasted_iota(jnp.int32, (BLK, A), 0)
    hh = (pix // W).astype(jnp.float32)
    ww = (pix % W).astype(jnp.float32)
    acx = (ww + 0.5) * STRIDE
    acy = (hh + 0.5) * STRIDE
    acol = lax.broadcasted_iota(jnp.int32, (BLK, A), 1)
    size = jnp.where(acol == 0, 32.0, jnp.where(acol == 1, 64.0, 128.0))
    dwc = jnp.minimum(dwv, SCALE_CLAMP)
    dhc = jnp.minimum(dhv, SCALE_CLAMP)
    pcx = dx * size + acx
    pcy = dy * size + acy
    pw = jnp.exp(dwc) * size
    ph = jnp.exp(dhc) * size
    sc_ref[...] = logits
    x1_ref[...] = jnp.clip(pcx - 0.5 * pw, 0.0, IMG)
    y1_ref[...] = jnp.clip(pcy - 0.5 * ph, 0.0, IMG)
    x2_ref[...] = jnp.clip(pcx + 0.5 * pw, 0.0, IMG)
    y2_ref[...] = jnp.clip(pcy + 0.5 * ph, 0.0, IMG)


def _k2_body(k_ref, a_ref, b_ref, c_ref, d_ref,
             ko_ref, ao_ref, bo_ref, co_ref, do_ref):
    key = k_ref[...]
    pls = [a_ref[...], b_ref[...], c_ref[...], d_ref[...]]
    key, pls = _bitonic_sort(key, pls, descending=True)
    rows = NTOP // 128
    ko_ref[...] = key[:rows]
    ao_ref[...] = pls[0][:rows]
    bo_ref[...] = pls[1][:rows]
    co_ref[...] = pls[2][:rows]
    do_ref[...] = pls[3][:rows]


def _k3_body(sk_ref, x1_ref, y1_ref, x2_ref, y2_ref,
             cx1_ref, cy1_ref, cx2_ref, cy2_ref,
             rx1_ref, ry1_ref, rx2_ref, ry2_ref,
             so_ref, x1o_ref, y1o_ref, x2o_ref, y2o_ref,
             s_scr):
    RT = NTOP // 128
    CHUNK = 128
    bx1 = rx1_ref[...]
    by1 = ry1_ref[...]
    bx2 = rx2_ref[...]
    by2 = ry2_ref[...]
    area_j = (bx2 - bx1) * (by2 - by1)
    jj = (lax.broadcasted_iota(jnp.int32, (CHUNK, RT, 128), 1) * 128
          + lax.broadcasted_iota(jnp.int32, (CHUNK, RT, 128), 2))
    ii_base = lax.broadcasted_iota(jnp.int32, (CHUNK, RT, 128), 0)
    for ck in range(NTOP // CHUNK):
        r0 = ck * CHUNK
        ax1 = cx1_ref[r0:r0 + CHUNK]
        ay1 = cy1_ref[r0:r0 + CHUNK]
        ax2 = cx2_ref[r0:r0 + CHUNK]
        ay2 = cy2_ref[r0:r0 + CHUNK]
        area_i = (ax2 - ax1) * (ay2 - ay1)
        ix1 = jnp.maximum(ax1, bx1)
        iy1 = jnp.maximum(ay1, by1)
        ix2 = jnp.minimum(ax2, bx2)
        iy2 = jnp.minimum(ay2, by2)
        inter = jnp.maximum(ix2 - ix1, 0.0) * jnp.maximum(iy2 - iy1, 0.0)
        iou = inter / (area_i + area_j - inter + 1e-9)
        ii = r0 + ii_base
        supp = ((iou > NMS_THRESH) & (jj > ii)
                & (ii < PRE_NMS_TOPK) & (jj < PRE_NMS_TOPK))
        s_scr[r0:r0 + CHUNK] = supp.astype(jnp.float32)

    flat = (lax.broadcasted_iota(jnp.int32, (RT, 128), 0) * 128
            + lax.broadcasted_iota(jnp.int32, (RT, 128), 1))

    def body(i, keep):
        row = s_scr[i]
        k_i = jnp.sum(jnp.where(flat == i, keep, 0.0))
        return keep * (1.0 - row * k_i)

    keep = lax.fori_loop(0, PRE_NMS_TOPK, body,
                         jnp.ones((RT, 128), jnp.float32))

    skey = sk_ref[...]
    masked = jnp.where(keep > 0.5, skey, jnp.float32(-1e9))
    flag = jnp.where(flat < PRE_NMS_TOPK,
                     jnp.where(keep > 0.5, 0, 1), 2)
    fkey = flag * NTOP * 2 + flat
    pls = [masked, x1_ref[...], y1_ref[...], x2_ref[...], y2_ref[...]]
    fkey, pls = _bitonic_sort(fkey, pls, descending=False)
    so_ref[...] = pls[0]
    x1o_ref[...] = pls[1]
    y1o_ref[...] = pls[2]
    x2o_ref[...] = pls[3]
    y2o_ref[...] = pls[4]


def kernel(features, W_conv, b_conv, W_obj, b_obj, W_delta, b_delta):
    f32 = jnp.float32
    x = features[0].transpose(1, 2, 0).reshape(NP, C)
    xpad = jnp.concatenate(
        [jnp.zeros((PADR, C), f32), x, jnp.zeros((PADR, C), f32)], axis=0)
    wtaps = W_conv.transpose(2, 3, 1, 0).reshape(9, C, C)
    bconv = b_conv.reshape(1, C)
    wobj = W_obj.reshape(A, C).T
    bobj = b_obj.reshape(1, A)
    wd = W_delta.reshape(A, 4, C)
    wdx = wd[:, 0, :].T
    wdy = wd[:, 1, :].T
    wdw = wd[:, 2, :].T
    wdh = wd[:, 3, :].T
    bd = b_delta.reshape(A, 4)
    bdx = bd[:, 0].reshape(1, A)
    bdy = bd[:, 1].reshape(1, A)
    bdw = bd[:, 2].reshape(1, A)
    bdh = bd[:, 3].reshape(1, A)

    o63 = jax.ShapeDtypeStruct((NP, A), f32)
    whole = lambda shape: pl.BlockSpec(shape, lambda i: (0,) * len(shape))
    sc, bx1, by1, bx2, by2 = pl.pallas_call(
        _k1_body,
        grid=(NP // BLK,),
        in_specs=[
            whole(xpad.shape),
            whole((9, C, C)),
            whole((1, C)),
            whole((C, A)), whole((1, A)),
            whole((C, A)), whole((C, A)), whole((C, A)), whole((C, A)),
            whole((1, A)), whole((1, A)), whole((1, A)), whole((1, A)),
        ],
        out_specs=[pl.BlockSpec((BLK, A), lambda i: (i, 0))] * 5,
        out_shape=(o63, o63, o63, o63, o63),
    )(xpad, wtaps, bconv, wobj, bobj, wdx, wdy, wdw, wdh,
      bdx, bdy, bdw, bdh)

    def padk(a, fill):
        return jnp.concatenate(
            [a.reshape(-1), jnp.full((NSORT - NANCH,), fill, f32)]
        ).reshape(NSORT // 128, 128)

    skey = padk(sc, NEG)
    sa = padk(bx1, 0.0)
    sb = padk(by1, 0.0)
    scd = padk(bx2, 0.0)
    sd = padk(by2, 0.0)

    RT = NTOP // 128
    ot = jax.ShapeDtypeStruct((RT, 128), f32)
    tk, tx1, ty1, tx2, ty2 = pl.pallas_call(
        _k2_body,
        out_shape=(ot, ot, ot, ot, ot),
    )(skey, sa, sb, scd, sd)

    def col(a):
        return a.reshape(NTOP, 1, 1)

    def row(a):
        return a.reshape(1, RT, 128)

    oi = jax.ShapeDtypeStruct((RT, 128), f32)
    fsc, fx1, fy1, fx2, fy2 = pl.pallas_call(
        _k3_body,
        out_shape=(oi, oi, oi, oi, oi),
        scratch_shapes=[pltpu.VMEM((NTOP, RT, 128), f32)],
    )(tk, tx1, ty1, tx2, ty2,
      col(tx1), col(ty1), col(tx2), col(ty2),
      row(tx1), row(ty1), row(tx2), row(ty2))

    fb = jnp.stack([fx1.reshape(-1)[:POST_NMS_TOPK],
                    fy1.reshape(-1)[:POST_NMS_TOPK],
                    fx2.reshape(-1)[:POST_NMS_TOPK],
                    fy2.reshape(-1)[:POST_NMS_TOPK]], axis=-1)
    fs = fsc.reshape(-1)[:POST_NMS_TOPK]
    return fb, fs
